# Initial kernel scaffold; baseline (speedup 1.0000x reference)
import jax
import jax.numpy as jnp
from jax.experimental import pallas as pl


def kernel(x, embedding_weight):
    raise NotImplementedError("write your pallas kernel here")


# speedup vs baseline: 1.0680x; 1.0680x over previous
import functools

import jax
import jax.numpy as jnp
from jax import lax
from jax.experimental import pallas as pl
from jax.experimental.pallas import tpu as pltpu
from jax.experimental.pallas import tpu_sc as plsc

B, S, D = 4, 2048, 1024
NC, NS, L = 2, 16, 16
NW = NC * NS
ROWS_PW = S // NW
CHUNK = 16
NCH = ROWS_PW // CHUNK
NXB = 3
NPB = 2

_mesh = plsc.VectorSubcoreMesh(
    core_axis_name="c", subcore_axis_name="s", num_cores=NC, num_subcores=NS
)


@functools.partial(
    pl.kernel,
    out_type=jax.ShapeDtypeStruct((B, S, D), jnp.float32),
    mesh=_mesh,
    scratch_types=[
        [pltpu.VMEM((CHUNK, D), jnp.float32) for _ in range(NXB)],
        [pltpu.VMEM((CHUNK, D), jnp.float32) for _ in range(NPB)],
        [pltpu.SemaphoreType.DMA for _ in range(NXB)],
        [pltpu.SemaphoreType.DMA for _ in range(NXB)],
        [pltpu.SemaphoreType.DMA for _ in range(NPB)],
    ],
)
def _pe_add(x_hbm, pe_hbm, out_hbm, xbufs, pbufs, sin, sout, spe):
    wid = lax.axis_index("s") * NC + lax.axis_index("c")
    base = wid * ROWS_PW

    T = NCH * B

    def start_in(t):
        c, b = t // B, t % B
        pltpu.async_copy(
            x_hbm.at[b, pl.ds(base + c * CHUNK, CHUNK), :],
            xbufs[t % NXB], sin[t % NXB])

    def wait_in(t):
        c, b = t // B, t % B
        pltpu.make_async_copy(
            x_hbm.at[b, pl.ds(base + c * CHUNK, CHUNK), :],
            xbufs[t % NXB], sin[t % NXB]).wait()

    def start_out(t):
        c, b = t // B, t % B
        pltpu.async_copy(
            xbufs[t % NXB],
            out_hbm.at[b, pl.ds(base + c * CHUNK, CHUNK), :],
            sout[t % NXB])

    def wait_out(t):
        c, b = t // B, t % B
        pltpu.make_async_copy(
            xbufs[t % NXB],
            out_hbm.at[b, pl.ds(base + c * CHUNK, CHUNK), :],
            sout[t % NXB]).wait()

    def start_pe(c):
        pltpu.async_copy(
            pe_hbm.at[pl.ds(base + c * CHUNK, CHUNK), :],
            pbufs[c % NPB], spe[c % NPB])

    def wait_pe(c):
        pltpu.make_async_copy(
            pe_hbm.at[pl.ds(base + c * CHUNK, CHUNK), :],
            pbufs[c % NPB], spe[c % NPB]).wait()

    start_pe(0)
    for t in range(NXB - 1):
        start_in(t)

    for t in range(T):
        c, b = t // B, t % B
        if b == 0:
            wait_pe(c)
            if c + 1 < NCH:
                start_pe(c + 1)
        wait_in(t)

        xb, pb = xbufs[t % NXB], pbufs[c % NPB]
        vecs_per_row = D // L

        @plsc.parallel_loop(0, CHUNK * vecs_per_row, 1, unroll=8)
        def _(k):
            r = k // vecs_per_row
            col = (k % vecs_per_row) * L
            plsc.addupdate(xb.at[r, pl.ds(col, L)], pb[r, pl.ds(col, L)])

        start_out(t)
        nxt = t + NXB - 1
        if nxt < T:
            if t >= 1:
                wait_out(t - 1)
            start_in(nxt)

    for t in range(max(0, T - NXB), T):
        wait_out(t)


def kernel(x, embedding_weight):
    return _pe_add(x, embedding_weight)


# speedup vs baseline: 1.1091x; 1.0384x over previous
import functools

import jax
import jax.numpy as jnp
from jax import lax
from jax.experimental import pallas as pl
from jax.experimental.pallas import tpu as pltpu
from jax.experimental.pallas import tpu_sc as plsc

B, S, D = 4, 2048, 1024
NC, NS, L = 2, 16, 16
NW = NC * NS
ROWS_PW = S // NW
CHUNK = 16
NCH = ROWS_PW // CHUNK
NXB = 5
NPB = 2

_mesh = plsc.VectorSubcoreMesh(
    core_axis_name="c", subcore_axis_name="s", num_cores=NC, num_subcores=NS
)


@functools.partial(
    pl.kernel,
    out_type=jax.ShapeDtypeStruct((B, S, D), jnp.float32),
    mesh=_mesh,
    scratch_types=[
        [pltpu.VMEM((CHUNK, D), jnp.float32) for _ in range(NXB)],
        [pltpu.VMEM((CHUNK, D), jnp.float32) for _ in range(NPB)],
        [pltpu.SemaphoreType.DMA for _ in range(NXB)],
        [pltpu.SemaphoreType.DMA for _ in range(NXB)],
        [pltpu.SemaphoreType.DMA for _ in range(NPB)],
    ],
)
def _pe_add(x_hbm, pe_hbm, out_hbm, xbufs, pbufs, sin, sout, spe):
    wid = lax.axis_index("s") * NC + lax.axis_index("c")
    base = wid * ROWS_PW

    T = NCH * B

    def start_in(t):
        c, b = t // B, t % B
        pltpu.async_copy(
            x_hbm.at[b, pl.ds(base + c * CHUNK, CHUNK), :],
            xbufs[t % NXB], sin[t % NXB])

    def wait_in(t):
        c, b = t // B, t % B
        pltpu.make_async_copy(
            x_hbm.at[b, pl.ds(base + c * CHUNK, CHUNK), :],
            xbufs[t % NXB], sin[t % NXB]).wait()

    def start_out(t):
        c, b = t // B, t % B
        pltpu.async_copy(
            xbufs[t % NXB],
            out_hbm.at[b, pl.ds(base + c * CHUNK, CHUNK), :],
            sout[t % NXB])

    def wait_out(t):
        c, b = t // B, t % B
        pltpu.make_async_copy(
            xbufs[t % NXB],
            out_hbm.at[b, pl.ds(base + c * CHUNK, CHUNK), :],
            sout[t % NXB]).wait()

    def start_pe(c):
        pltpu.async_copy(
            pe_hbm.at[pl.ds(base + c * CHUNK, CHUNK), :],
            pbufs[c % NPB], spe[c % NPB])

    def wait_pe(c):
        pltpu.make_async_copy(
            pe_hbm.at[pl.ds(base + c * CHUNK, CHUNK), :],
            pbufs[c % NPB], spe[c % NPB]).wait()

    start_pe(0)
    for t in range(NXB - 1):
        start_in(t)

    for t in range(T):
        c, b = t // B, t % B
        if b == 0:
            wait_pe(c)
            if c + 1 < NCH:
                start_pe(c + 1)
        wait_in(t)

        xb, pb = xbufs[t % NXB], pbufs[c % NPB]
        vecs_per_row = D // L

        @plsc.parallel_loop(0, CHUNK * vecs_per_row, 1, unroll=8)
        def _(k):
            r = k // vecs_per_row
            col = (k % vecs_per_row) * L
            plsc.addupdate(xb.at[r, pl.ds(col, L)], pb[r, pl.ds(col, L)])

        start_out(t)
        nxt = t + NXB - 1
        if nxt < T:
            if t >= 1:
                wait_out(t - 1)
            start_in(nxt)

    for t in range(max(0, T - NXB), T):
        wait_out(t)


def kernel(x, embedding_weight):
    return _pe_add(x, embedding_weight)
